# Initial kernel scaffold; baseline (speedup 1.0000x reference)
import jax
import jax.numpy as jnp
from jax.experimental import pallas as pl


def kernel(x, edge_index, W):
    raise NotImplementedError("write your pallas kernel here")


# speedup vs baseline: 26.0440x; 26.0440x over previous
import functools

import jax
import jax.numpy as jnp
from jax import lax
from jax.experimental import pallas as pl
from jax.experimental.pallas import tpu as pltpu
from jax.experimental.pallas import tpu_sc as plsc

NC = 2
NS = 16
NW = NC * NS


def _deg_call(n, epw):
    @functools.partial(
        pl.kernel,
        out_type=jax.ShapeDtypeStruct((NW, n), jnp.float32),
        mesh=plsc.VectorSubcoreMesh(core_axis_name="c", subcore_axis_name="s"),
        scratch_types=[
            pltpu.VMEM((epw,), jnp.int32),
            pltpu.VMEM((n,), jnp.float32),
        ],
        compiler_params=pltpu.CompilerParams(needs_layout_passes=False),
    )
    def deg_kernel(dst_hbm, out_hbm, idx_v, hist_v):
        c = lax.axis_index("c")
        s = lax.axis_index("s")
        wid = c * NS + s
        pltpu.sync_copy(dst_hbm.at[wid], idx_v)
        zeros16 = jnp.zeros((16,), jnp.float32)

        def zbody(i, carry):
            hist_v[pl.ds(i * 16, 16)] = zeros16
            return carry

        lax.fori_loop(0, n // 16, zbody, 0)
        ones16 = jnp.ones((16,), jnp.float32)

        def body(i, carry):
            idx16 = idx_v[pl.ds(i * 16, 16)]
            plsc.addupdate_scatter(hist_v, [idx16], ones16)
            return carry

        lax.fori_loop(0, epw // 16, body, 0)
        pltpu.sync_copy(hist_v, out_hbm.at[wid])

    return deg_kernel


def _agg_call(n, d, nch, ch, np_):
    rpt = np_ // NS

    @functools.partial(
        pl.kernel,
        out_type=jax.ShapeDtypeStruct((NC, np_, d), jnp.float32),
        mesh=plsc.VectorSubcoreMesh(core_axis_name="c", subcore_axis_name="s"),
        scratch_types=[
            pltpu.VMEM((nch, ch), jnp.int32),
            pltpu.VMEM((nch, ch), jnp.int32),
            pltpu.VMEM((ch, d), jnp.float32),
            pltpu.VMEM_SHARED((np_, d), jnp.float32),
        ],
        compiler_params=pltpu.CompilerParams(needs_layout_passes=False),
    )
    def agg_kernel(src_hbm, dst_hbm, g_hbm, out_hbm,
                   src_v, dst_v, rows_v, acc_sh):
        c = lax.axis_index("c")
        s = lax.axis_index("s")
        wid = c * NS + s
        pltpu.sync_copy(src_hbm.at[wid], src_v)
        pltpu.sync_copy(dst_hbm.at[wid], dst_v)

        zeros16 = jnp.zeros((16,), jnp.float32)
        vpr = d // 16

        def zb(i, carry):
            rows_v[i // vpr, pl.ds((i % vpr) * 16, 16)] = zeros16
            return carry

        lax.fori_loop(0, ch * vpr, zb, 0)

        def zc(i, carry):
            pltpu.sync_copy(rows_v, acc_sh.at[pl.ds(s * rpt + i * ch, ch)])
            return carry

        lax.fori_loop(0, rpt // ch, zc, 0)
        plsc.subcore_barrier()

        def body(j, carry):
            pltpu.sync_copy(g_hbm.at[src_v.at[j]], rows_v)
            pltpu.sync_copy(rows_v, acc_sh.at[dst_v.at[j]], add=True)
            return carry

        lax.fori_loop(0, nch, body, 0)
        plsc.subcore_barrier()
        pltpu.sync_copy(acc_sh.at[pl.ds(s * rpt, rpt)],
                        out_hbm.at[c, pl.ds(s * rpt, rpt)])

    return agg_kernel


def _tc1_body(x_ref, w_ref, degp_ref, g_ref):
    deg = jnp.sum(degp_ref[...], axis=0)
    inv = lax.rsqrt(jnp.maximum(deg, 1.0))
    h = lax.dot_general(x_ref[...], w_ref[...], (((1,), (1,)), ((), ())),
                        preferred_element_type=jnp.float32)
    g_ref[...] = h * inv[:, None]


def _tc2_body(p_ref, degp_ref, o_ref):
    deg = jnp.sum(degp_ref[...], axis=0)
    inv = lax.rsqrt(jnp.maximum(deg, 1.0))
    o_ref[...] = (p_ref[0, :deg.shape[0]] + p_ref[1, :deg.shape[0]]) * inv[:, None]


def kernel(x, edge_index, W):
    n, d = x.shape
    e = edge_index.shape[1]
    epw = e // NW
    ch = 80
    nch = epw // ch
    quantum = NS * ch
    while quantum % 8:
        quantum *= 2
    np_ = ((n + quantum - 1) // quantum) * quantum

    src_r = edge_index[0].reshape(NW, nch, ch)
    dst_r = edge_index[1].reshape(NW, nch, ch)
    dst_flat = edge_index[1].reshape(NW, epw)

    degp = _deg_call(n, epw)(dst_flat)

    g = pl.pallas_call(
        _tc1_body,
        out_shape=jax.ShapeDtypeStruct((n, d), jnp.float32),
    )(x, W, degp)

    partials = _agg_call(n, d, nch, ch, np_)(src_r, dst_r, g)

    out = pl.pallas_call(
        _tc2_body,
        out_shape=jax.ShapeDtypeStruct((n, d), jnp.float32),
    )(partials, degp)

    return out


# speedup vs baseline: 31.2704x; 1.2007x over previous
import functools

import jax
import jax.numpy as jnp
from jax import lax
from jax.experimental import pallas as pl
from jax.experimental.pallas import tpu as pltpu
from jax.experimental.pallas import tpu_sc as plsc

NC = 2
NS = 16
NW = NC * NS


def _deg_call(n, epw):
    @functools.partial(
        pl.kernel,
        out_type=jax.ShapeDtypeStruct((NW, n), jnp.float32),
        mesh=plsc.VectorSubcoreMesh(core_axis_name="c", subcore_axis_name="s"),
        scratch_types=[
            pltpu.VMEM((epw,), jnp.int32),
            pltpu.VMEM((n,), jnp.float32),
        ],
        compiler_params=pltpu.CompilerParams(needs_layout_passes=False),
    )
    def deg_kernel(dst_hbm, out_hbm, idx_v, hist_v):
        c = lax.axis_index("c")
        s = lax.axis_index("s")
        wid = c * NS + s
        pltpu.sync_copy(dst_hbm.at[wid], idx_v)
        zeros16 = jnp.zeros((16,), jnp.float32)

        def zbody(i, carry):
            hist_v[pl.ds(i * 16, 16)] = zeros16
            return carry

        lax.fori_loop(0, n // 16, zbody, 0)
        ones16 = jnp.ones((16,), jnp.float32)

        def body(i, carry):
            idx16 = idx_v[pl.ds(i * 16, 16)]
            plsc.addupdate_scatter(hist_v, [idx16], ones16)
            return carry

        lax.fori_loop(0, epw // 16, body, 0)
        pltpu.sync_copy(hist_v, out_hbm.at[wid])

    return deg_kernel


def _agg_call(n, d, nch, ch, np_, gc):
    ng = nch // gc
    rpt = np_ // NS

    @functools.partial(
        pl.kernel,
        out_type=jax.ShapeDtypeStruct((NC, np_, d), jnp.float32),
        mesh=plsc.VectorSubcoreMesh(core_axis_name="c", subcore_axis_name="s"),
        scratch_types=[
            pltpu.VMEM((gc, ch), jnp.int32),
            pltpu.VMEM((gc, ch), jnp.int32),
            pltpu.VMEM((2, ch, d), jnp.float32),
            pltpu.VMEM_SHARED((np_, d), jnp.float32),
            pltpu.SemaphoreType.DMA,
        ],
        compiler_params=pltpu.CompilerParams(needs_layout_passes=False),
    )
    def agg_kernel(src_hbm, dst_hbm, g_hbm, out_hbm,
                   src_v, dst_v, rows_v, acc_sh, sem):
        c = lax.axis_index("c")
        s = lax.axis_index("s")
        wid = c * NS + s

        zeros16 = jnp.zeros((16,), jnp.float32)
        vpr = d // 16

        def zb(i, carry):
            rows_v[0, i // vpr, pl.ds((i % vpr) * 16, 16)] = zeros16
            return carry

        lax.fori_loop(0, ch * vpr, zb, 0)

        def zc(i, carry):
            pltpu.sync_copy(rows_v.at[0],
                            acc_sh.at[pl.ds(s * rpt + i * ch, ch)])
            return carry

        lax.fori_loop(0, rpt // ch, zc, 0)
        plsc.subcore_barrier()

        for grp in range(ng):
            pltpu.sync_copy(src_hbm.at[wid, grp], src_v)
            pltpu.sync_copy(dst_hbm.at[wid, grp], dst_v)
            pltpu.async_copy(g_hbm.at[src_v.at[0]], rows_v.at[0], sem)

            def body(j, carry):
                b = lax.rem(j, 2)
                pltpu.make_async_copy(g_hbm.at[src_v.at[j]],
                                      rows_v.at[b], sem).wait()

                @pl.when(j + 1 < gc)
                def _():
                    pltpu.async_copy(g_hbm.at[src_v.at[j + 1]],
                                     rows_v.at[1 - b], sem)

                pltpu.sync_copy(rows_v.at[b], acc_sh.at[dst_v.at[j]],
                                add=True)
                return carry

            lax.fori_loop(0, gc, body, 0)
        plsc.subcore_barrier()
        pltpu.sync_copy(acc_sh.at[pl.ds(s * rpt, rpt)],
                        out_hbm.at[c, pl.ds(s * rpt, rpt)])

    return agg_kernel


def _tc1_body(x_ref, w_ref, degp_ref, g_ref):
    deg = jnp.sum(degp_ref[...], axis=0)
    inv = lax.rsqrt(jnp.maximum(deg, 1.0))
    h = lax.dot_general(x_ref[...], w_ref[...], (((1,), (1,)), ((), ())),
                        preferred_element_type=jnp.float32)
    g_ref[...] = h * inv[:, None]


def _tc2_body(p_ref, degp_ref, o_ref):
    deg = jnp.sum(degp_ref[...], axis=0)
    inv = lax.rsqrt(jnp.maximum(deg, 1.0))
    o_ref[...] = (p_ref[0, :deg.shape[0]] + p_ref[1, :deg.shape[0]]) * inv[:, None]


def kernel(x, edge_index, W):
    n, d = x.shape
    e = edge_index.shape[1]
    epw = e // NW
    ch = 80
    nch = epw // ch
    gc = 25
    quantum = NS * ch
    while quantum % 8:
        quantum *= 2
    np_ = ((n + quantum - 1) // quantum) * quantum

    src_r = edge_index[0].reshape(NW, nch // gc, gc, ch)
    dst_r = edge_index[1].reshape(NW, nch // gc, gc, ch)
    dst_flat = edge_index[1].reshape(NW, epw)

    degp = _deg_call(n, epw)(dst_flat)

    g = pl.pallas_call(
        _tc1_body,
        out_shape=jax.ShapeDtypeStruct((n, d), jnp.float32),
    )(x, W, degp)

    partials = _agg_call(n, d, nch, ch, np_, gc)(src_r, dst_r, g)

    out = pl.pallas_call(
        _tc2_body,
        out_shape=jax.ShapeDtypeStruct((n, d), jnp.float32),
    )(partials, degp)

    return out


# speedup vs baseline: 37.2575x; 1.1915x over previous
import functools

import jax
import jax.numpy as jnp
from jax import lax
from jax.experimental import pallas as pl
from jax.experimental.pallas import tpu as pltpu
from jax.experimental.pallas import tpu_sc as plsc

NC = 2
NS = 16
NW = NC * NS


def _deg_call(n, epw):
    @functools.partial(
        pl.kernel,
        out_type=jax.ShapeDtypeStruct((NW, n), jnp.float32),
        mesh=plsc.VectorSubcoreMesh(core_axis_name="c", subcore_axis_name="s"),
        scratch_types=[
            pltpu.VMEM((epw,), jnp.int32),
            pltpu.VMEM((n,), jnp.float32),
        ],
        compiler_params=pltpu.CompilerParams(needs_layout_passes=False),
    )
    def deg_kernel(dst_hbm, out_hbm, idx_v, hist_v):
        c = lax.axis_index("c")
        s = lax.axis_index("s")
        wid = c * NS + s
        pltpu.sync_copy(dst_hbm.at[wid], idx_v)
        zeros16 = jnp.zeros((16,), jnp.float32)

        def zbody(i, carry):
            hist_v[pl.ds(i * 16, 16)] = zeros16
            return carry

        lax.fori_loop(0, n // 16, zbody, 0)
        ones16 = jnp.ones((16,), jnp.float32)

        def body(i, carry):
            idx16 = idx_v[pl.ds(i * 16, 16)]
            plsc.addupdate_scatter(hist_v, [idx16], ones16)
            return carry

        lax.fori_loop(0, epw // 16, body, 0)
        pltpu.sync_copy(hist_v, out_hbm.at[wid])

    return deg_kernel


def _agg_call(n, d, nch, ch, np_, gc):
    ng = nch // gc
    rpt = np_ // NS

    @functools.partial(
        pl.kernel,
        out_type=jax.ShapeDtypeStruct((NC, np_, d), jnp.float32),
        mesh=plsc.VectorSubcoreMesh(core_axis_name="c", subcore_axis_name="s"),
        scratch_types=[
            pltpu.VMEM((gc, ch), jnp.int32),
            pltpu.VMEM((gc, ch), jnp.int32),
            pltpu.VMEM((2, ch, d), jnp.float32),
            pltpu.VMEM_SHARED((np_, d), jnp.float32),
            pltpu.SemaphoreType.DMA,
            pltpu.SemaphoreType.DMA,
        ],
        compiler_params=pltpu.CompilerParams(needs_layout_passes=False),
    )
    def agg_kernel(src_hbm, dst_hbm, g_hbm, out_hbm,
                   src_v, dst_v, rows_v, acc_sh, sem_a, sem_b):
        c = lax.axis_index("c")
        s = lax.axis_index("s")
        wid = c * NS + s

        zeros16 = jnp.zeros((16,), jnp.float32)
        vpr = d // 16

        def zb(i, carry):
            rows_v[0, i // vpr, pl.ds((i % vpr) * 16, 16)] = zeros16
            return carry

        lax.fori_loop(0, ch * vpr, zb, 0)

        def zc(i, carry):
            pltpu.sync_copy(rows_v.at[0],
                            acc_sh.at[pl.ds(s * rpt + i * ch, ch)])
            return carry

        lax.fori_loop(0, rpt // ch, zc, 0)
        plsc.subcore_barrier()

        for grp in range(ng):
            pltpu.sync_copy(src_hbm.at[wid, grp], src_v)
            pltpu.sync_copy(dst_hbm.at[wid, grp], dst_v)
            pltpu.async_copy(g_hbm.at[src_v.at[0]], rows_v.at[0], sem_a)
            pltpu.async_copy(g_hbm.at[src_v.at[1]], rows_v.at[1], sem_b)

            def pair(k, carry):
                ja = 2 * k
                jb = 2 * k + 1
                pltpu.make_async_copy(g_hbm.at[src_v.at[ja]],
                                      rows_v.at[0], sem_a).wait()
                pltpu.sync_copy(rows_v.at[0], acc_sh.at[dst_v.at[ja]],
                                add=True)

                @pl.when(ja + 2 < gc)
                def _():
                    pltpu.async_copy(g_hbm.at[src_v.at[ja + 2]],
                                     rows_v.at[0], sem_a)

                pltpu.make_async_copy(g_hbm.at[src_v.at[jb]],
                                      rows_v.at[1], sem_b).wait()
                pltpu.sync_copy(rows_v.at[1], acc_sh.at[dst_v.at[jb]],
                                add=True)

                @pl.when(jb + 2 < gc)
                def _():
                    pltpu.async_copy(g_hbm.at[src_v.at[jb + 2]],
                                     rows_v.at[1], sem_b)

                return carry

            lax.fori_loop(0, gc // 2, pair, 0)
            if gc % 2:
                pltpu.make_async_copy(g_hbm.at[src_v.at[gc - 1]],
                                      rows_v.at[0], sem_a).wait()
                pltpu.sync_copy(rows_v.at[0], acc_sh.at[dst_v.at[gc - 1]],
                                add=True)
        plsc.subcore_barrier()
        pltpu.sync_copy(acc_sh.at[pl.ds(s * rpt, rpt)],
                        out_hbm.at[c, pl.ds(s * rpt, rpt)])

    return agg_kernel


def _tc1_body(x_ref, w_ref, degp_ref, g_ref):
    deg = jnp.sum(degp_ref[...], axis=0)
    inv = lax.rsqrt(jnp.maximum(deg, 1.0))
    h = lax.dot_general(x_ref[...], w_ref[...], (((1,), (1,)), ((), ())),
                        preferred_element_type=jnp.float32)
    g_ref[...] = h * inv[:, None]


def _tc2_body(p_ref, degp_ref, o_ref):
    deg = jnp.sum(degp_ref[...], axis=0)
    inv = lax.rsqrt(jnp.maximum(deg, 1.0))
    o_ref[...] = (p_ref[0, :deg.shape[0]] + p_ref[1, :deg.shape[0]]) * inv[:, None]


def kernel(x, edge_index, W):
    n, d = x.shape
    e = edge_index.shape[1]
    epw = e // NW
    ch = 80
    nch = epw // ch
    gc = 25
    quantum = NS * ch
    while quantum % 8:
        quantum *= 2
    np_ = ((n + quantum - 1) // quantum) * quantum

    src_r = edge_index[0].reshape(NW, nch // gc, gc, ch)
    dst_r = edge_index[1].reshape(NW, nch // gc, gc, ch)
    dst_flat = edge_index[1].reshape(NW, epw)

    degp = _deg_call(n, epw)(dst_flat)

    g = pl.pallas_call(
        _tc1_body,
        out_shape=jax.ShapeDtypeStruct((n, d), jnp.float32),
    )(x, W, degp)

    partials = _agg_call(n, d, nch, ch, np_, gc)(src_r, dst_r, g)

    out = pl.pallas_call(
        _tc2_body,
        out_shape=jax.ShapeDtypeStruct((n, d), jnp.float32),
    )(partials, degp)

    return out


# speedup vs baseline: 41.5628x; 1.1156x over previous
import functools

import jax
import jax.numpy as jnp
from jax import lax
from jax.experimental import pallas as pl
from jax.experimental.pallas import tpu as pltpu
from jax.experimental.pallas import tpu_sc as plsc

NC = 2
NS = 16
NW = NC * NS


def _deg_call(n, epw):
    @functools.partial(
        pl.kernel,
        out_type=jax.ShapeDtypeStruct((NW, n), jnp.float32),
        mesh=plsc.VectorSubcoreMesh(core_axis_name="c", subcore_axis_name="s"),
        scratch_types=[
            pltpu.VMEM((epw,), jnp.int32),
            pltpu.VMEM((n,), jnp.float32),
        ],
        compiler_params=pltpu.CompilerParams(needs_layout_passes=False),
    )
    def deg_kernel(dst_hbm, out_hbm, idx_v, hist_v):
        c = lax.axis_index("c")
        s = lax.axis_index("s")
        wid = c * NS + s
        pltpu.sync_copy(dst_hbm.at[wid], idx_v)
        zeros16 = jnp.zeros((16,), jnp.float32)

        def zbody(i, carry):
            hist_v[pl.ds(i * 16, 16)] = zeros16
            return carry

        lax.fori_loop(0, n // 16, zbody, 0)
        ones16 = jnp.ones((16,), jnp.float32)

        def body(i, carry):
            idx16 = idx_v[pl.ds(i * 16, 16)]
            plsc.addupdate_scatter(hist_v, [idx16], ones16)
            return carry

        lax.fori_loop(0, epw // 16, body, 0)
        pltpu.sync_copy(hist_v, out_hbm.at[wid])

    return deg_kernel


def _agg_call(n, d, nch, ch, np_, gc):
    ng = nch // gc
    rpt = np_ // NS

    @functools.partial(
        pl.kernel,
        out_type=jax.ShapeDtypeStruct((NC, np_, d), jnp.float32),
        mesh=plsc.VectorSubcoreMesh(core_axis_name="c", subcore_axis_name="s"),
        scratch_types=[
            pltpu.VMEM((gc, ch), jnp.int32),
            pltpu.VMEM((gc, ch), jnp.int32),
            pltpu.VMEM((3, ch, d), jnp.float32),
            pltpu.VMEM_SHARED((np_, d), jnp.float32),
            pltpu.SemaphoreType.DMA,
            pltpu.SemaphoreType.DMA,
            pltpu.SemaphoreType.DMA,
        ],
        compiler_params=pltpu.CompilerParams(needs_layout_passes=False),
    )
    def agg_kernel(src_hbm, dst_hbm, g_hbm, out_hbm,
                   src_v, dst_v, rows_v, acc_sh, sem_a, sem_b, sem_c):
        c = lax.axis_index("c")
        s = lax.axis_index("s")
        wid = c * NS + s

        zeros16 = jnp.zeros((16,), jnp.float32)
        vpr = d // 16

        def zb(i, carry):
            rows_v[0, i // vpr, pl.ds((i % vpr) * 16, 16)] = zeros16
            return carry

        lax.fori_loop(0, ch * vpr, zb, 0)

        def zc(i, carry):
            pltpu.sync_copy(rows_v.at[0],
                            acc_sh.at[pl.ds(s * rpt + i * ch, ch)])
            return carry

        lax.fori_loop(0, rpt // ch, zc, 0)
        plsc.subcore_barrier()

        nb = 3
        sems = (sem_a, sem_b, sem_c)
        for grp in range(ng):
            pltpu.sync_copy(src_hbm.at[wid, grp], src_v)
            pltpu.sync_copy(dst_hbm.at[wid, grp], dst_v)
            for t in range(nb):
                pltpu.async_copy(g_hbm.at[src_v.at[t]], rows_v.at[t],
                                 sems[t])

            def body(k, carry):
                for t in range(nb):
                    j = nb * k + t
                    pltpu.make_async_copy(g_hbm.at[src_v.at[j]],
                                          rows_v.at[t], sems[t]).wait()
                    pltpu.sync_copy(rows_v.at[t], acc_sh.at[dst_v.at[j]],
                                    add=True)

                    @pl.when(j + nb < gc)
                    def _():
                        pltpu.async_copy(g_hbm.at[src_v.at[j + nb]],
                                         rows_v.at[t], sems[t])

                return carry

            lax.fori_loop(0, gc // nb, body, 0)
            for j in range(nb * (gc // nb), gc):
                t = j % nb
                pltpu.make_async_copy(g_hbm.at[src_v.at[j]],
                                      rows_v.at[t], sems[t]).wait()
                pltpu.sync_copy(rows_v.at[t], acc_sh.at[dst_v.at[j]],
                                add=True)
        plsc.subcore_barrier()
        pltpu.sync_copy(acc_sh.at[pl.ds(s * rpt, rpt)],
                        out_hbm.at[c, pl.ds(s * rpt, rpt)])

    return agg_kernel


def _tc1_body(x_ref, w_ref, degp_ref, g_ref):
    deg = jnp.sum(degp_ref[...], axis=0)
    inv = lax.rsqrt(jnp.maximum(deg, 1.0))
    h = lax.dot_general(x_ref[...], w_ref[...], (((1,), (1,)), ((), ())),
                        preferred_element_type=jnp.float32)
    g_ref[...] = h * inv[:, None]


def _tc2_body(p_ref, degp_ref, o_ref):
    deg = jnp.sum(degp_ref[...], axis=0)
    inv = lax.rsqrt(jnp.maximum(deg, 1.0))
    o_ref[...] = (p_ref[0, :deg.shape[0]] + p_ref[1, :deg.shape[0]]) * inv[:, None]


def kernel(x, edge_index, W):
    n, d = x.shape
    e = edge_index.shape[1]
    epw = e // NW
    ch = 80
    nch = epw // ch
    gc = 25
    quantum = NS * ch
    while quantum % 8:
        quantum *= 2
    np_ = ((n + quantum - 1) // quantum) * quantum

    src_r = edge_index[0].reshape(NW, nch // gc, gc, ch)
    dst_r = edge_index[1].reshape(NW, nch // gc, gc, ch)
    dst_flat = edge_index[1].reshape(NW, epw)

    degp = _deg_call(n, epw)(dst_flat)

    g = pl.pallas_call(
        _tc1_body,
        out_shape=jax.ShapeDtypeStruct((n, d), jnp.float32),
    )(x, W, degp)

    partials = _agg_call(n, d, nch, ch, np_, gc)(src_r, dst_r, g)

    out = pl.pallas_call(
        _tc2_body,
        out_shape=jax.ShapeDtypeStruct((n, d), jnp.float32),
    )(partials, degp)

    return out
